# Initial kernel scaffold; baseline (speedup 1.0000x reference)
import jax
import jax.numpy as jnp
from jax.experimental import pallas as pl


def kernel(feat, edge_index, WC, WD, proj_W, proj_b):
    raise NotImplementedError("write your pallas kernel here")


# speedup vs baseline: 1.1472x; 1.1472x over previous
import functools

import jax
import jax.numpy as jnp
from jax import lax
from jax.experimental import pallas as pl
from jax.experimental.pallas import tpu as pltpu
from jax.experimental.pallas import tpu_sc as plsc

N_NODES = 10000
N_EDGES = 160000
IN_CH = 256
HID = 256

NC = 2
NS = 16
NW = NC * NS

EP = 163840
EPW = EP // NW
C2 = 64
NCH2 = EPW // C2

EPT = EP // NS
C4 = 128
NCH4 = EPT // C4

NP = 10240
ROWS_PER_TILE = NP // NS
ZROWS = 128

_mesh = plsc.VectorSubcoreMesh(core_axis_name="c", subcore_axis_name="s")


def _k1_body(feat_ref, w_ref, zq_ref, zcat_ref):
    blk = lax.dot_general(feat_ref[...], w_ref[0],
                          (((1,), (1,)), ((), ())),
                          preferred_element_type=jnp.float32)
    zq_ref[0] = blk
    zcat_ref[...] = blk


def _k1(feat, wcat4):
    return pl.pallas_call(
        _k1_body,
        grid=(5, 4),
        in_specs=[
            pl.BlockSpec((2000, IN_CH), lambda i, q: (i, 0)),
            pl.BlockSpec((1, 128, IN_CH), lambda i, q: (q, 0, 0)),
        ],
        out_specs=[
            pl.BlockSpec((1, 2000, 128), lambda i, q: (q, i, 0)),
            pl.BlockSpec((2000, 128), lambda i, q: (i, q)),
        ],
        out_shape=[
            jax.ShapeDtypeStruct((4, N_NODES, 128), jnp.float32),
            jax.ShapeDtypeStruct((N_NODES, 512), jnp.float32),
        ],
    )(feat, wcat4)


def _k2_body(zcat, srcp, dstp, sco_hbm, sdi_hbm, sidx, didx, srows, drows,
             sco, sdi, sem):
    wid = lax.axis_index("c") * NS + lax.axis_index("s")
    wbase = wid * EPW

    def chunk_body(j, _):
        base = wbase + j * C2
        pltpu.sync_copy(srcp.at[pl.ds(base, C2)], sidx)
        pltpu.sync_copy(dstp.at[pl.ds(base, C2)], didx)
        c1 = pltpu.async_copy(zcat.at[sidx], srows, sem)
        c2 = pltpu.async_copy(zcat.at[didx], drows, sem)
        c1.wait()
        c2.wait()

        lane = lax.broadcasted_iota(jnp.int32, (16,), 0)

        def group_body(g, _):
            sc_c = jnp.zeros((16,), jnp.float32)
            sc_d = jnp.zeros((16,), jnp.float32)
            for i in range(16):
                e = g * 16 + i
                acc_c = jnp.zeros((16,), jnp.float32)
                acc_d = jnp.zeros((16,), jnp.float32)
                for k in range(16):
                    sv = srows[e, pl.ds(k * 16, 16)]
                    dv = drows[e, pl.ds(k * 16, 16)]
                    acc_c = acc_c + sv * dv
                for k in range(16, 32):
                    sv = srows[e, pl.ds(k * 16, 16)]
                    dv = drows[e, pl.ds(k * 16, 16)]
                    df = sv - dv
                    acc_d = acc_d + df * df
                sc_c = jnp.where(lane == i, jnp.sum(acc_c), sc_c)
                sc_d = jnp.where(lane == i, jnp.sum(acc_d), sc_d)
            sco[pl.ds(j * C2 + g * 16, 16)] = sc_c
            sdi[pl.ds(j * C2 + g * 16, 16)] = sc_d
            return 0

        lax.fori_loop(0, C2 // 16, group_body, 0)
        return 0

    lax.fori_loop(0, NCH2, chunk_body, 0)
    pltpu.sync_copy(sco, sco_hbm.at[pl.ds(wbase, EPW)])
    pltpu.sync_copy(sdi, sdi_hbm.at[pl.ds(wbase, EPW)])


_k2 = functools.partial(
    pl.kernel,
    out_type=(jax.ShapeDtypeStruct((EP,), jnp.float32),
              jax.ShapeDtypeStruct((EP,), jnp.float32)),
    mesh=_mesh,
    scratch_types=[
        pltpu.VMEM((C2,), jnp.int32),
        pltpu.VMEM((C2,), jnp.int32),
        pltpu.VMEM((C2, 512), jnp.float32),
        pltpu.VMEM((C2, 512), jnp.float32),
        pltpu.VMEM((EPW,), jnp.float32),
        pltpu.VMEM((EPW,), jnp.float32),
        pltpu.SemaphoreType.DMA,
    ],
    compiler_params=pltpu.CompilerParams(needs_layout_passes=False),
)(_k2_body)


def _k3_body(co_ref, di_ref, ao_ref, ad_ref):
    row = lax.broadcasted_iota(jnp.int32, (EP // 128, 128), 0)
    valid = row < (N_EDGES // 128)
    for ref, oref in ((co_ref, ao_ref), (di_ref, ad_ref)):
        x = ref[...]
        xm = jnp.where(valid, x, -jnp.inf)
        m = jnp.max(xm)
        ex = jnp.where(valid, jnp.exp(x - m), 0.0)
        oref[...] = ex / jnp.sum(ex)


def _k3(sco, sdi):
    out = pl.pallas_call(
        _k3_body,
        out_shape=[jax.ShapeDtypeStruct((EP // 128, 128), jnp.float32),
                   jax.ShapeDtypeStruct((EP // 128, 128), jnp.float32)],
    )(sco.reshape(EP // 128, 128), sdi.reshape(EP // 128, 128))
    return out[0].reshape(EP), out[1].reshape(EP)


def _k4_body(zq, srcp, dstp, alpha_c, alpha_d, out, sidx, didx2, gidx, alph,
             rows, zbuf, acc, sem):
    c = lax.axis_index("c")
    s = lax.axis_index("s")

    def zb_body(r, _):
        for k in range(8):
            zbuf[r, pl.ds(k * 16, 16)] = jnp.zeros((16,), jnp.float32)
        return 0

    lax.fori_loop(0, ZROWS, zb_body, 0)

    for p, alpha in ((0, alpha_c), (1, alpha_d)):
        qoff = (2 * p + c) * N_NODES
        for i in range(5):
            pltpu.sync_copy(zbuf, acc.at[pl.ds(s * ROWS_PER_TILE + i * ZROWS,
                                               ZROWS)])
        plsc.subcore_barrier()

        def chunk_body(j, _):
            base = s * EPT + j * C4
            pltpu.sync_copy(srcp.at[pl.ds(base, C4)], sidx)
            pltpu.sync_copy(dstp.at[pl.ds(base, C4)], didx2.at[0])
            pltpu.sync_copy(alpha.at[pl.ds(base, C4)], alph)
            for k in range(8):
                gidx[pl.ds(k * 16, 16)] = sidx[pl.ds(k * 16, 16)] + qoff
            pltpu.async_copy(zq.at[gidx], rows, sem).wait()

            def g_body(g, _):
                av = alph[pl.ds(g * 16, 16)]
                for i in range(16):
                    e = g * 16 + i
                    a = av[i]
                    for k in range(8):
                        rows[e, pl.ds(k * 16, 16)] = (
                            rows[e, pl.ds(k * 16, 16)] * a)
                return 0

            lax.fori_loop(0, C4 // 16, g_body, 0)
            pltpu.sync_copy(rows, acc.at[didx2.at[0]], add=True)
            return 0

        lax.fori_loop(0, NCH4, chunk_body, 0)
        plsc.subcore_barrier()
        pltpu.sync_copy(acc.at[pl.ds(s * ROWS_PER_TILE, ROWS_PER_TILE)],
                        out.at[2 * p + c, pl.ds(s * ROWS_PER_TILE,
                                                ROWS_PER_TILE)])
        plsc.subcore_barrier()


_k4 = functools.partial(
    pl.kernel,
    out_type=jax.ShapeDtypeStruct((4, NP, 128), jnp.float32),
    mesh=_mesh,
    scratch_types=[
        pltpu.VMEM((C4,), jnp.int32),
        pltpu.VMEM((1, C4), jnp.int32),
        pltpu.VMEM((C4,), jnp.int32),
        pltpu.VMEM((C4,), jnp.float32),
        pltpu.VMEM((C4, 128), jnp.float32),
        pltpu.VMEM((ZROWS, 128), jnp.float32),
        pltpu.VMEM_SHARED((NP, 128), jnp.float32),
        pltpu.SemaphoreType.DMA,
    ],
    compiler_params=pltpu.CompilerParams(needs_layout_passes=False),
)(_k4_body)


def _k5_body(zagg_ref, w4_ref, b_ref, o_ref):
    acc = jnp.broadcast_to(b_ref[0], (2000, HID))
    for q in range(4):
        acc = acc + lax.dot_general(zagg_ref[q], w4_ref[q],
                                    (((1,), (1,)), ((), ())),
                                    preferred_element_type=jnp.float32)
    o_ref[...] = acc


def _k5(zagg, w4, b):
    return pl.pallas_call(
        _k5_body,
        grid=(5,),
        in_specs=[
            pl.BlockSpec((4, 2000, 128), lambda i: (0, i, 0)),
            pl.BlockSpec((4, HID, 128), lambda i: (0, 0, 0)),
            pl.BlockSpec((1, HID), lambda i: (0, 0)),
        ],
        out_specs=pl.BlockSpec((2000, HID), lambda i: (i, 0)),
        out_shape=jax.ShapeDtypeStruct((N_NODES, HID), jnp.float32),
    )(zagg, w4, b)


def kernel(feat, edge_index, WC, WD, proj_W, proj_b):
    src = edge_index[0].astype(jnp.int32)
    dst = edge_index[1].astype(jnp.int32)
    pad = jnp.zeros((EP - N_EDGES,), jnp.int32)
    srcp = jnp.concatenate([src, pad])
    dstp = jnp.concatenate([dst, pad])

    wcat4 = jnp.concatenate([WC, WD], axis=0).reshape(4, 128, IN_CH)
    zq, zcat = _k1(feat, wcat4)
    zq2 = zq.reshape(4 * N_NODES, 128)

    sco, sdi = _k2(zcat, srcp, dstp)
    alpha_c, alpha_d = _k3(sco, sdi)
    zagg = _k4(zq2, srcp, dstp, alpha_c, alpha_d)

    w4 = proj_W.reshape(HID, 4, 128).transpose(1, 0, 2)
    b2 = proj_b.reshape(1, HID)
    return _k5(zagg, w4, b2)


# speedup vs baseline: 1.4528x; 1.2664x over previous
import functools

import jax
import jax.numpy as jnp
from jax import lax
from jax.experimental import pallas as pl
from jax.experimental.pallas import tpu as pltpu
from jax.experimental.pallas import tpu_sc as plsc

N_NODES = 10000
N_EDGES = 160000
IN_CH = 256
HID = 256

NC = 2
NS = 16
NW = NC * NS

EP = 163840
EPW = EP // NW
C2 = 32
NCH2 = EPW // C2

EPT = EP // NS
C4 = 64
NCH4 = EPT // C4

NP = 10240
ROWS_PER_TILE = NP // NS
ZROWS = 128

_mesh = plsc.VectorSubcoreMesh(core_axis_name="c", subcore_axis_name="s")


def _k1_body(feat_ref, w_ref, zq_ref, zcat_ref):
    blk = lax.dot_general(feat_ref[...], w_ref[0],
                          (((1,), (1,)), ((), ())),
                          preferred_element_type=jnp.float32)
    zq_ref[0] = blk
    zcat_ref[...] = blk


def _k1(feat, wcat4):
    return pl.pallas_call(
        _k1_body,
        grid=(5, 4),
        in_specs=[
            pl.BlockSpec((2000, IN_CH), lambda i, q: (i, 0)),
            pl.BlockSpec((1, 128, IN_CH), lambda i, q: (q, 0, 0)),
        ],
        out_specs=[
            pl.BlockSpec((1, 2000, 128), lambda i, q: (q, i, 0)),
            pl.BlockSpec((2000, 128), lambda i, q: (i, q)),
        ],
        out_shape=[
            jax.ShapeDtypeStruct((4, N_NODES, 128), jnp.float32),
            jax.ShapeDtypeStruct((N_NODES, 512), jnp.float32),
        ],
    )(feat, wcat4)


def _k2_body(zcat, srcp, dstp, sco_hbm, sdi_hbm, sidx_all, didx_all, srows,
             drows, sco, sdi, sem0, sem1):
    wid = lax.axis_index("c") * NS + lax.axis_index("s")
    wbase = wid * EPW
    pltpu.sync_copy(srcp.at[pl.ds(wbase, EPW)], sidx_all)
    pltpu.sync_copy(dstp.at[pl.ds(wbase, EPW)], didx_all)
    lane = lax.broadcasted_iota(jnp.int32, (16,), 0)

    def issue(j, slot, sem):
        pltpu.async_copy(zcat.at[sidx_all.at[pl.ds(j * C2, C2)]],
                         srows.at[slot], sem)
        pltpu.async_copy(zcat.at[didx_all.at[pl.ds(j * C2, C2)]],
                         drows.at[slot], sem)

    def wait_slot(slot, sem):
        pltpu.make_async_copy(zcat.at[sidx_all.at[pl.ds(0, C2)]],
                              srows.at[slot], sem).wait()
        pltpu.make_async_copy(zcat.at[didx_all.at[pl.ds(0, C2)]],
                              drows.at[slot], sem).wait()

    def compute(j, slot):
        def group_body(g, _):
            sc_c = jnp.zeros((16,), jnp.float32)
            sc_d = jnp.zeros((16,), jnp.float32)
            for i in range(16):
                e = g * 16 + i
                acc_c = jnp.zeros((16,), jnp.float32)
                acc_d = jnp.zeros((16,), jnp.float32)
                for k in range(16):
                    sv = srows[slot, e, pl.ds(k * 16, 16)]
                    dv = drows[slot, e, pl.ds(k * 16, 16)]
                    acc_c = acc_c + sv * dv
                for k in range(16, 32):
                    sv = srows[slot, e, pl.ds(k * 16, 16)]
                    dv = drows[slot, e, pl.ds(k * 16, 16)]
                    df = sv - dv
                    acc_d = acc_d + df * df
                sc_c = jnp.where(lane == i, jnp.sum(acc_c), sc_c)
                sc_d = jnp.where(lane == i, jnp.sum(acc_d), sc_d)
            sco[pl.ds(j * C2 + g * 16, 16)] = sc_c
            sdi[pl.ds(j * C2 + g * 16, 16)] = sc_d
            return 0

        lax.fori_loop(0, C2 // 16, group_body, 0)

    issue(0, 0, sem0)

    def pair_body(m, _):
        j0 = 2 * m
        issue(j0 + 1, 1, sem1)
        wait_slot(0, sem0)
        compute(j0, 0)
        issue(jnp.minimum(j0 + 2, NCH2 - 1), 0, sem0)
        wait_slot(1, sem1)
        compute(j0 + 1, 1)
        return 0

    lax.fori_loop(0, NCH2 // 2, pair_body, 0)
    wait_slot(0, sem0)
    pltpu.sync_copy(sco, sco_hbm.at[pl.ds(wbase, EPW)])
    pltpu.sync_copy(sdi, sdi_hbm.at[pl.ds(wbase, EPW)])


_k2 = functools.partial(
    pl.kernel,
    out_type=(jax.ShapeDtypeStruct((EP,), jnp.float32),
              jax.ShapeDtypeStruct((EP,), jnp.float32)),
    mesh=_mesh,
    scratch_types=[
        pltpu.VMEM((EPW,), jnp.int32),
        pltpu.VMEM((EPW,), jnp.int32),
        pltpu.VMEM((2, C2, 512), jnp.float32),
        pltpu.VMEM((2, C2, 512), jnp.float32),
        pltpu.VMEM((EPW,), jnp.float32),
        pltpu.VMEM((EPW,), jnp.float32),
        pltpu.SemaphoreType.DMA,
        pltpu.SemaphoreType.DMA,
    ],
    compiler_params=pltpu.CompilerParams(needs_layout_passes=False),
)(_k2_body)


def _k3_body(co_ref, di_ref, ao_ref, ad_ref):
    row = lax.broadcasted_iota(jnp.int32, (EP // 128, 128), 0)
    valid = row < (N_EDGES // 128)
    for ref, oref in ((co_ref, ao_ref), (di_ref, ad_ref)):
        x = ref[...]
        xm = jnp.where(valid, x, -jnp.inf)
        m = jnp.max(xm)
        ex = jnp.where(valid, jnp.exp(x - m), 0.0)
        oref[...] = ex / jnp.sum(ex)


def _k3(sco, sdi):
    out = pl.pallas_call(
        _k3_body,
        out_shape=[jax.ShapeDtypeStruct((EP // 128, 128), jnp.float32),
                   jax.ShapeDtypeStruct((EP // 128, 128), jnp.float32)],
    )(sco.reshape(EP // 128, 128), sdi.reshape(EP // 128, 128))
    return out[0].reshape(EP), out[1].reshape(EP)


def _k4_body(zq, srcp, dstp2d, alpha_c, alpha_d, out, didx_all, alph_all,
             sidxb, gidxb, rows, acc, semi0, semi1, semr0, semr1):
    c = lax.axis_index("c")
    s = lax.axis_index("s")
    pltpu.sync_copy(dstp2d.at[pl.ds(s * NCH4, NCH4)], didx_all)
    semi = (semi0, semi1)
    semr = (semr0, semr1)

    for p, alpha in ((0, alpha_c), (1, alpha_d)):
        qoff = (2 * p + c) * N_NODES
        pltpu.sync_copy(alpha.at[pl.ds(s * EPT, EPT)], alph_all)

        def zb_body(r, _):
            for k in range(8):
                rows[0, r, pl.ds(k * 16, 16)] = jnp.zeros((16,), jnp.float32)
            return 0

        lax.fori_loop(0, C4, zb_body, 0)
        for i in range(ROWS_PER_TILE // C4):
            pltpu.sync_copy(rows.at[0], acc.at[pl.ds(s * ROWS_PER_TILE
                                                     + i * C4, C4)])
        plsc.subcore_barrier()

        def fetch_idx(j, slot):
            pltpu.async_copy(srcp.at[pl.ds(s * EPT + j * C4, C4)],
                             sidxb.at[slot], semi[slot])

        def body(j, _):
            slot = j % 2
            other = 1 - slot

            @pl.when(j < NCH4)
            def _():
                for sl in range(2):
                    @pl.when(slot == sl)
                    def _():
                        pltpu.make_async_copy(
                            srcp.at[pl.ds(0, C4)], sidxb.at[sl],
                            semi[sl]).wait()
                        for k in range(C4 // 16):
                            gidxb[sl, pl.ds(k * 16, 16)] = (
                                sidxb[sl, pl.ds(k * 16, 16)] + qoff)
                        pltpu.async_copy(zq.at[gidxb.at[sl]], rows.at[sl],
                                        semr[sl])

                        @pl.when(j + 1 < NCH4)
                        def _():
                            fetch_idx(j + 1, 1 - sl)

            @pl.when(j > 0)
            def _():
                jm = j - 1
                for sl in range(2):
                    @pl.when(other == sl)
                    def _():
                        pltpu.make_async_copy(zq.at[gidxb.at[sl]],
                                              rows.at[sl], semr[sl]).wait()

                        def g_body(g, _):
                            av = alph_all[pl.ds(jm * C4 + g * 16, 16)]
                            for i in range(16):
                                e = g * 16 + i
                                a = av[i]
                                for k in range(8):
                                    rows[sl, e, pl.ds(k * 16, 16)] = (
                                        rows[sl, e, pl.ds(k * 16, 16)] * a)
                            return 0

                        lax.fori_loop(0, C4 // 16, g_body, 0)
                        pltpu.sync_copy(rows.at[sl], acc.at[didx_all.at[jm]],
                                        add=True)
            return 0

        fetch_idx(0, 0)
        lax.fori_loop(0, NCH4 + 1, body, 0)
        plsc.subcore_barrier()
        pltpu.sync_copy(acc.at[pl.ds(s * ROWS_PER_TILE, ROWS_PER_TILE)],
                        out.at[2 * p + c, pl.ds(s * ROWS_PER_TILE,
                                                ROWS_PER_TILE)])
        plsc.subcore_barrier()


_k4 = functools.partial(
    pl.kernel,
    out_type=jax.ShapeDtypeStruct((4, NP, 128), jnp.float32),
    mesh=_mesh,
    scratch_types=[
        pltpu.VMEM((NCH4, C4), jnp.int32),
        pltpu.VMEM((EPT,), jnp.float32),
        pltpu.VMEM((2, C4), jnp.int32),
        pltpu.VMEM((2, C4), jnp.int32),
        pltpu.VMEM((2, C4, 128), jnp.float32),
        pltpu.VMEM_SHARED((NP, 128), jnp.float32),
        pltpu.SemaphoreType.DMA,
        pltpu.SemaphoreType.DMA,
        pltpu.SemaphoreType.DMA,
        pltpu.SemaphoreType.DMA,
    ],
    compiler_params=pltpu.CompilerParams(needs_layout_passes=False),
)(_k4_body)


def _k5_body(zagg_ref, w4_ref, b_ref, o_ref):
    acc = jnp.broadcast_to(b_ref[0], (2000, HID))
    for q in range(4):
        acc = acc + lax.dot_general(zagg_ref[q], w4_ref[q],
                                    (((1,), (1,)), ((), ())),
                                    preferred_element_type=jnp.float32)
    o_ref[...] = acc


def _k5(zagg, w4, b):
    return pl.pallas_call(
        _k5_body,
        grid=(5,),
        in_specs=[
            pl.BlockSpec((4, 2000, 128), lambda i: (0, i, 0)),
            pl.BlockSpec((4, HID, 128), lambda i: (0, 0, 0)),
            pl.BlockSpec((1, HID), lambda i: (0, 0)),
        ],
        out_specs=pl.BlockSpec((2000, HID), lambda i: (i, 0)),
        out_shape=jax.ShapeDtypeStruct((N_NODES, HID), jnp.float32),
    )(zagg, w4, b)


def kernel(feat, edge_index, WC, WD, proj_W, proj_b):
    src = edge_index[0].astype(jnp.int32)
    dst = edge_index[1].astype(jnp.int32)
    pad = jnp.zeros((EP - N_EDGES,), jnp.int32)
    srcp = jnp.concatenate([src, pad])
    dstp = jnp.concatenate([dst, pad])

    wcat4 = jnp.concatenate([WC, WD], axis=0).reshape(4, 128, IN_CH)
    zq, zcat = _k1(feat, wcat4)
    zq2 = zq.reshape(4 * N_NODES, 128)

    sco, sdi = _k2(zcat, srcp, dstp)
    alpha_c, alpha_d = _k3(sco, sdi)
    zagg = _k4(zq2, srcp, dstp.reshape(EP // C4, C4), alpha_c, alpha_d)

    w4 = proj_W.reshape(HID, 4, 128).transpose(1, 0, 2)
    b2 = proj_b.reshape(1, HID)
    return _k5(zagg, w4, b2)


# speedup vs baseline: 1.4941x; 1.0284x over previous
import functools

import jax
import jax.numpy as jnp
from jax import lax
from jax.experimental import pallas as pl
from jax.experimental.pallas import tpu as pltpu
from jax.experimental.pallas import tpu_sc as plsc

N_NODES = 10000
N_EDGES = 160000
IN_CH = 256
HID = 256

NC = 2
NS = 16
NW = NC * NS

EP = 163840
EPW = EP // NW
C2 = 32
NCH2 = EPW // C2

EPT = EP // NS
C4 = 64
NCH4 = EPT // C4

NP = 10240
ROWS_PER_TILE = NP // NS
ZROWS = 128

_mesh = plsc.VectorSubcoreMesh(core_axis_name="c", subcore_axis_name="s")


def _k1_body(feat_ref, w_ref, zq_ref, zcat_ref):
    blk = lax.dot_general(feat_ref[...], w_ref[0],
                          (((1,), (1,)), ((), ())),
                          preferred_element_type=jnp.float32)
    zq_ref[0] = blk
    zcat_ref[...] = blk


def _k1(feat, wcat4):
    return pl.pallas_call(
        _k1_body,
        grid=(5, 4),
        in_specs=[
            pl.BlockSpec((2000, IN_CH), lambda i, q: (i, 0)),
            pl.BlockSpec((1, 128, IN_CH), lambda i, q: (q, 0, 0)),
        ],
        out_specs=[
            pl.BlockSpec((1, 2000, 128), lambda i, q: (q, i, 0)),
            pl.BlockSpec((2000, 128), lambda i, q: (i, q)),
        ],
        out_shape=[
            jax.ShapeDtypeStruct((4, N_NODES, 128), jnp.float32),
            jax.ShapeDtypeStruct((N_NODES, 512), jnp.float32),
        ],
    )(feat, wcat4)


def _k2_body(zcat, srcp, dstp, sco_hbm, sdi_hbm, sidx_all, didx_all, srows,
             drows, sco, sdi, sem0, sem1):
    wid = lax.axis_index("c") * NS + lax.axis_index("s")
    wbase = wid * EPW
    pltpu.sync_copy(srcp.at[pl.ds(wbase, EPW)], sidx_all)
    pltpu.sync_copy(dstp.at[pl.ds(wbase, EPW)], didx_all)
    lane = lax.broadcasted_iota(jnp.int32, (16,), 0)

    def issue(j, slot, sem):
        pltpu.async_copy(zcat.at[sidx_all.at[pl.ds(j * C2, C2)]],
                         srows.at[slot], sem)
        pltpu.async_copy(zcat.at[didx_all.at[pl.ds(j * C2, C2)]],
                         drows.at[slot], sem)

    def wait_slot(slot, sem):
        pltpu.make_async_copy(zcat.at[sidx_all.at[pl.ds(0, C2)]],
                              srows.at[slot], sem).wait()
        pltpu.make_async_copy(zcat.at[didx_all.at[pl.ds(0, C2)]],
                              drows.at[slot], sem).wait()

    def compute(j, slot):
        def group_body(g, _):
            sc_c = jnp.zeros((16,), jnp.float32)
            sc_d = jnp.zeros((16,), jnp.float32)
            for i in range(16):
                e = g * 16 + i
                ac = [jnp.zeros((16,), jnp.float32) for _ in range(4)]
                ad = [jnp.zeros((16,), jnp.float32) for _ in range(4)]
                for k in range(16):
                    sv = srows[slot, e, pl.ds(k * 16, 16)]
                    dv = drows[slot, e, pl.ds(k * 16, 16)]
                    ac[k % 4] = ac[k % 4] + sv * dv
                for k in range(16, 32):
                    sv = srows[slot, e, pl.ds(k * 16, 16)]
                    dv = drows[slot, e, pl.ds(k * 16, 16)]
                    df = sv - dv
                    ad[k % 4] = ad[k % 4] + df * df
                acc_c = (ac[0] + ac[1]) + (ac[2] + ac[3])
                acc_d = (ad[0] + ad[1]) + (ad[2] + ad[3])
                sc_c = jnp.where(lane == i, jnp.sum(acc_c), sc_c)
                sc_d = jnp.where(lane == i, jnp.sum(acc_d), sc_d)
            sco[pl.ds(j * C2 + g * 16, 16)] = sc_c
            sdi[pl.ds(j * C2 + g * 16, 16)] = sc_d
            return 0

        lax.fori_loop(0, C2 // 16, group_body, 0)

    issue(0, 0, sem0)

    def pair_body(m, _):
        j0 = 2 * m
        issue(j0 + 1, 1, sem1)
        wait_slot(0, sem0)
        compute(j0, 0)
        issue(jnp.minimum(j0 + 2, NCH2 - 1), 0, sem0)
        wait_slot(1, sem1)
        compute(j0 + 1, 1)
        return 0

    lax.fori_loop(0, NCH2 // 2, pair_body, 0)
    wait_slot(0, sem0)
    pltpu.sync_copy(sco, sco_hbm.at[pl.ds(wbase, EPW)])
    pltpu.sync_copy(sdi, sdi_hbm.at[pl.ds(wbase, EPW)])


_k2 = functools.partial(
    pl.kernel,
    out_type=(jax.ShapeDtypeStruct((EP,), jnp.float32),
              jax.ShapeDtypeStruct((EP,), jnp.float32)),
    mesh=_mesh,
    scratch_types=[
        pltpu.VMEM((EPW,), jnp.int32),
        pltpu.VMEM((EPW,), jnp.int32),
        pltpu.VMEM((2, C2, 512), jnp.float32),
        pltpu.VMEM((2, C2, 512), jnp.float32),
        pltpu.VMEM((EPW,), jnp.float32),
        pltpu.VMEM((EPW,), jnp.float32),
        pltpu.SemaphoreType.DMA,
        pltpu.SemaphoreType.DMA,
    ],
    compiler_params=pltpu.CompilerParams(needs_layout_passes=False),
)(_k2_body)


def _k3_body(co_ref, di_ref, ao_ref, ad_ref):
    row = lax.broadcasted_iota(jnp.int32, (EP // 128, 128), 0)
    valid = row < (N_EDGES // 128)
    for ref, oref in ((co_ref, ao_ref), (di_ref, ad_ref)):
        x = ref[...]
        xm = jnp.where(valid, x, -jnp.inf)
        m = jnp.max(xm)
        ex = jnp.where(valid, jnp.exp(x - m), 0.0)
        oref[...] = ex / jnp.sum(ex)


def _k3(sco, sdi):
    out = pl.pallas_call(
        _k3_body,
        out_shape=[jax.ShapeDtypeStruct((EP // 128, 128), jnp.float32),
                   jax.ShapeDtypeStruct((EP // 128, 128), jnp.float32)],
    )(sco.reshape(EP // 128, 128), sdi.reshape(EP // 128, 128))
    return out[0].reshape(EP), out[1].reshape(EP)


def _k4_body(zq, srcp, dstp2d, alpha_c, alpha_d, out, didx_all, alph_all,
             sidxb, gidxb, rows, acc, semi0, semi1, semr0, semr1, sems0,
             sems1):
    c = lax.axis_index("c")
    s = lax.axis_index("s")
    pltpu.sync_copy(dstp2d.at[pl.ds(s * NCH4, NCH4)], didx_all)
    semi = (semi0, semi1)
    semr = (semr0, semr1)
    sems = (sems0, sems1)

    for p, alpha in ((0, alpha_c), (1, alpha_d)):
        qoff = (2 * p + c) * N_NODES
        pltpu.sync_copy(alpha.at[pl.ds(s * EPT, EPT)], alph_all)

        def zb_body(r, _):
            for k in range(8):
                rows[0, r, pl.ds(k * 16, 16)] = jnp.zeros((16,), jnp.float32)
            return 0

        lax.fori_loop(0, C4, zb_body, 0)
        for i in range(ROWS_PER_TILE // C4):
            pltpu.sync_copy(rows.at[0], acc.at[pl.ds(s * ROWS_PER_TILE
                                                     + i * C4, C4)])
        plsc.subcore_barrier()

        def fetch_idx(j, slot):
            pltpu.async_copy(srcp.at[pl.ds(s * EPT + j * C4, C4)],
                             sidxb.at[slot], semi[slot])

        def body(j, _):
            slot = j % 2
            other = 1 - slot

            @pl.when(j < NCH4)
            def _():
                for sl in range(2):
                    @pl.when(slot == sl)
                    def _():
                        pltpu.make_async_copy(
                            srcp.at[pl.ds(0, C4)], sidxb.at[sl],
                            semi[sl]).wait()
                        for k in range(C4 // 16):
                            gidxb[sl, pl.ds(k * 16, 16)] = (
                                sidxb[sl, pl.ds(k * 16, 16)] + qoff)

                        @pl.when(j >= 2)
                        def _():
                            pltpu.make_async_copy(
                                rows.at[sl], acc.at[didx_all.at[0]],
                                sems[sl]).wait()

                        pltpu.async_copy(zq.at[gidxb.at[sl]], rows.at[sl],
                                        semr[sl])

                        @pl.when(j + 1 < NCH4)
                        def _():
                            fetch_idx(j + 1, 1 - sl)

            @pl.when(j > 0)
            def _():
                jm = j - 1
                for sl in range(2):
                    @pl.when(other == sl)
                    def _():
                        pltpu.make_async_copy(zq.at[gidxb.at[sl]],
                                              rows.at[sl], semr[sl]).wait()

                        def g_body(g, _):
                            av = alph_all[pl.ds(jm * C4 + g * 16, 16)]
                            for i in range(16):
                                e = g * 16 + i
                                a = av[i]
                                for k in range(8):
                                    rows[sl, e, pl.ds(k * 16, 16)] = (
                                        rows[sl, e, pl.ds(k * 16, 16)] * a)
                            return 0

                        lax.fori_loop(0, C4 // 16, g_body, 0)
                        pltpu.async_copy(rows.at[sl], acc.at[didx_all.at[jm]],
                                         sems[sl], add=True)
            return 0

        fetch_idx(0, 0)
        lax.fori_loop(0, NCH4 + 1, body, 0)
        for sl in range(2):
            pltpu.make_async_copy(rows.at[sl], acc.at[didx_all.at[0]],
                                  sems[sl]).wait()
        plsc.subcore_barrier()
        pltpu.sync_copy(acc.at[pl.ds(s * ROWS_PER_TILE, ROWS_PER_TILE)],
                        out.at[2 * p + c, pl.ds(s * ROWS_PER_TILE,
                                                ROWS_PER_TILE)])
        plsc.subcore_barrier()


_k4 = functools.partial(
    pl.kernel,
    out_type=jax.ShapeDtypeStruct((4, NP, 128), jnp.float32),
    mesh=_mesh,
    scratch_types=[
        pltpu.VMEM((NCH4, C4), jnp.int32),
        pltpu.VMEM((EPT,), jnp.float32),
        pltpu.VMEM((2, C4), jnp.int32),
        pltpu.VMEM((2, C4), jnp.int32),
        pltpu.VMEM((2, C4, 128), jnp.float32),
        pltpu.VMEM_SHARED((NP, 128), jnp.float32),
        pltpu.SemaphoreType.DMA,
        pltpu.SemaphoreType.DMA,
        pltpu.SemaphoreType.DMA,
        pltpu.SemaphoreType.DMA,
        pltpu.SemaphoreType.DMA,
        pltpu.SemaphoreType.DMA,
    ],
    compiler_params=pltpu.CompilerParams(needs_layout_passes=False),
)(_k4_body)


def _k5_body(zagg_ref, w4_ref, b_ref, o_ref):
    acc = jnp.broadcast_to(b_ref[0], (2000, HID))
    for q in range(4):
        acc = acc + lax.dot_general(zagg_ref[q], w4_ref[q],
                                    (((1,), (1,)), ((), ())),
                                    preferred_element_type=jnp.float32)
    o_ref[...] = acc


def _k5(zagg, w4, b):
    return pl.pallas_call(
        _k5_body,
        grid=(5,),
        in_specs=[
            pl.BlockSpec((4, 2000, 128), lambda i: (0, i, 0)),
            pl.BlockSpec((4, HID, 128), lambda i: (0, 0, 0)),
            pl.BlockSpec((1, HID), lambda i: (0, 0)),
        ],
        out_specs=pl.BlockSpec((2000, HID), lambda i: (i, 0)),
        out_shape=jax.ShapeDtypeStruct((N_NODES, HID), jnp.float32),
    )(zagg, w4, b)


def kernel(feat, edge_index, WC, WD, proj_W, proj_b):
    src = edge_index[0].astype(jnp.int32)
    dst = edge_index[1].astype(jnp.int32)
    pad = jnp.zeros((EP - N_EDGES,), jnp.int32)
    srcp = jnp.concatenate([src, pad])
    dstp = jnp.concatenate([dst, pad])

    wcat4 = jnp.concatenate([WC, WD], axis=0).reshape(4, 128, IN_CH)
    zq, zcat = _k1(feat, wcat4)
    zq2 = zq.reshape(4 * N_NODES, 128)

    sco, sdi = _k2(zcat, srcp, dstp)
    alpha_c, alpha_d = _k3(sco, sdi)
    zagg = _k4(zq2, srcp, dstp.reshape(EP // C4, C4), alpha_c, alpha_d)

    w4 = proj_W.reshape(HID, 4, 128).transpose(1, 0, 2)
    b2 = proj_b.reshape(1, HID)
    return _k5(zagg, w4, b2)


# speedup vs baseline: 1.5195x; 1.0170x over previous
import functools

import jax
import jax.numpy as jnp
from jax import lax
from jax.experimental import pallas as pl
from jax.experimental.pallas import tpu as pltpu
from jax.experimental.pallas import tpu_sc as plsc

N_NODES = 10000
N_EDGES = 160000
IN_CH = 256
HID = 256

NC = 2
NS = 16
NW = NC * NS

EP = 163840
EPW = EP // NW
C2 = 32
NCH2 = EPW // C2

EPT = EP // NS
C4 = 64
NCH4 = EPT // C4

NP = 10240
ROWS_PER_TILE = NP // NS
ZROWS = 128

_mesh = plsc.VectorSubcoreMesh(core_axis_name="c", subcore_axis_name="s")


def _k1_body(feat_ref, w_ref, zq_ref):
    zq_ref[0] = lax.dot_general(feat_ref[...], w_ref[0],
                                (((1,), (1,)), ((), ())),
                                preferred_element_type=jnp.float32)


def _k1(feat, wcat4):
    return pl.pallas_call(
        _k1_body,
        grid=(5, 4),
        in_specs=[
            pl.BlockSpec((2000, IN_CH), lambda i, q: (i, 0)),
            pl.BlockSpec((1, 128, IN_CH), lambda i, q: (q, 0, 0)),
        ],
        out_specs=pl.BlockSpec((1, 2000, 128), lambda i, q: (q, i, 0)),
        out_shape=jax.ShapeDtypeStruct((4, N_NODES, 128), jnp.float32),
    )(feat, wcat4)


def _k2_body(zq, srcp2d, dstp2d, sco_hbm, sdi_hbm, sidxb, didxb, gidx, srows,
             drows, sco, sdi, semi0, semi1, semr0, semr1):
    wid = lax.axis_index("c") * NS + lax.axis_index("s")
    wbase = wid * EPW
    lane = lax.broadcasted_iota(jnp.int32, (16,), 0)
    semi = (semi0, semi1)
    semr = (semr0, semr1)

    def fetch_idx(j, slot):
        pltpu.async_copy(srcp2d.at[wid * NCH2 + j], sidxb.at[slot],
                         semi[slot])
        pltpu.async_copy(dstp2d.at[wid * NCH2 + j], didxb.at[slot],
                         semi[slot])

    def body(j, _):
        slot = j % 2
        other = 1 - slot

        @pl.when(j < NCH2)
        def _():
            for sl in range(2):
                @pl.when(slot == sl)
                def _():
                    pltpu.make_async_copy(srcp2d.at[0], sidxb.at[sl],
                                          semi[sl]).wait()
                    pltpu.make_async_copy(dstp2d.at[0], didxb.at[sl],
                                          semi[sl]).wait()
                    for q in range(4):
                        for k in range(C2 // 16):
                            ks = pl.ds(k * 16, 16)
                            gidx[sl, q, ks] = (sidxb[sl, ks]
                                               + (q * N_NODES))
                            gidx[sl, 4 + q, ks] = (didxb[sl, ks]
                                                   + (q * N_NODES))
                    for q in range(4):
                        pltpu.async_copy(
                            zq.at[gidx.at[sl, q]],
                            srows.at[sl].at[pl.ds(q * C2, C2)], semr[sl])
                        pltpu.async_copy(
                            zq.at[gidx.at[sl, 4 + q]],
                            drows.at[sl].at[pl.ds(q * C2, C2)], semr[sl])

                    @pl.when(j + 1 < NCH2)
                    def _():
                        fetch_idx(j + 1, 1 - sl)

        @pl.when(j > 0)
        def _():
            jm = j - 1
            for sl in range(2):
                @pl.when(other == sl)
                def _():
                    pltpu.make_async_copy(zq.at[gidx.at[sl, 0]],
                                          srows.at[sl], semr[sl]).wait()
                    pltpu.make_async_copy(zq.at[gidx.at[sl, 0]],
                                          drows.at[sl], semr[sl]).wait()

                    def group_body(g, _):
                        sc_c = jnp.zeros((16,), jnp.float32)
                        sc_d = jnp.zeros((16,), jnp.float32)
                        for i in range(16):
                            e = g * 16 + i
                            ac = [jnp.zeros((16,), jnp.float32)
                                  for _ in range(4)]
                            ad = [jnp.zeros((16,), jnp.float32)
                                  for _ in range(4)]
                            for q in range(2):
                                for k in range(8):
                                    ks = pl.ds(k * 16, 16)
                                    sv = srows[sl, q * C2 + e, ks]
                                    dv = drows[sl, q * C2 + e, ks]
                                    ac[k % 4] = ac[k % 4] + sv * dv
                            for q in range(2, 4):
                                for k in range(8):
                                    ks = pl.ds(k * 16, 16)
                                    sv = srows[sl, q * C2 + e, ks]
                                    dv = drows[sl, q * C2 + e, ks]
                                    df = sv - dv
                                    ad[k % 4] = ad[k % 4] + df * df
                            acc_c = (ac[0] + ac[1]) + (ac[2] + ac[3])
                            acc_d = (ad[0] + ad[1]) + (ad[2] + ad[3])
                            sc_c = jnp.where(lane == i, jnp.sum(acc_c), sc_c)
                            sc_d = jnp.where(lane == i, jnp.sum(acc_d), sc_d)
                        sco[pl.ds(jm * C2 + g * 16, 16)] = sc_c
                        sdi[pl.ds(jm * C2 + g * 16, 16)] = sc_d
                        return 0

                    lax.fori_loop(0, C2 // 16, group_body, 0)
        return 0

    fetch_idx(0, 0)
    lax.fori_loop(0, NCH2 + 1, body, 0)
    pltpu.sync_copy(sco, sco_hbm.at[pl.ds(wbase, EPW)])
    pltpu.sync_copy(sdi, sdi_hbm.at[pl.ds(wbase, EPW)])


_k2 = functools.partial(
    pl.kernel,
    out_type=(jax.ShapeDtypeStruct((EP,), jnp.float32),
              jax.ShapeDtypeStruct((EP,), jnp.float32)),
    mesh=_mesh,
    scratch_types=[
        pltpu.VMEM((2, C2), jnp.int32),
        pltpu.VMEM((2, C2), jnp.int32),
        pltpu.VMEM((2, 8, C2), jnp.int32),
        pltpu.VMEM((2, 4 * C2, 128), jnp.float32),
        pltpu.VMEM((2, 4 * C2, 128), jnp.float32),
        pltpu.VMEM((EPW,), jnp.float32),
        pltpu.VMEM((EPW,), jnp.float32),
        pltpu.SemaphoreType.DMA,
        pltpu.SemaphoreType.DMA,
        pltpu.SemaphoreType.DMA,
        pltpu.SemaphoreType.DMA,
    ],
    compiler_params=pltpu.CompilerParams(needs_layout_passes=False),
)(_k2_body)


def _k3_body(co_ref, di_ref, ao_ref, ad_ref):
    row = lax.broadcasted_iota(jnp.int32, (EP // 128, 128), 0)
    valid = row < (N_EDGES // 128)
    for ref, oref in ((co_ref, ao_ref), (di_ref, ad_ref)):
        x = ref[...]
        xm = jnp.where(valid, x, -jnp.inf)
        m = jnp.max(xm)
        ex = jnp.where(valid, jnp.exp(x - m), 0.0)
        oref[...] = ex / jnp.sum(ex)


def _k3(sco, sdi):
    out = pl.pallas_call(
        _k3_body,
        out_shape=[jax.ShapeDtypeStruct((EP // 128, 128), jnp.float32),
                   jax.ShapeDtypeStruct((EP // 128, 128), jnp.float32)],
    )(sco.reshape(EP // 128, 128), sdi.reshape(EP // 128, 128))
    return out[0].reshape(EP), out[1].reshape(EP)


def _k4_body(zq, srcp, dstp2d, alpha_c, alpha_d, out, didx_all, alph_all,
             sidxb, gidxb, rows, acc, semi0, semi1, semr0, semr1, sems0,
             sems1):
    c = lax.axis_index("c")
    s = lax.axis_index("s")
    pltpu.sync_copy(dstp2d.at[pl.ds(s * NCH4, NCH4)], didx_all)
    semi = (semi0, semi1)
    semr = (semr0, semr1)
    sems = (sems0, sems1)

    for p, alpha in ((0, alpha_c), (1, alpha_d)):
        qoff = (2 * p + c) * N_NODES
        pltpu.sync_copy(alpha.at[pl.ds(s * EPT, EPT)], alph_all)

        def zb_body(r, _):
            for k in range(8):
                rows[0, r, pl.ds(k * 16, 16)] = jnp.zeros((16,), jnp.float32)
            return 0

        lax.fori_loop(0, C4, zb_body, 0)
        for i in range(ROWS_PER_TILE // C4):
            pltpu.sync_copy(rows.at[0], acc.at[pl.ds(s * ROWS_PER_TILE
                                                     + i * C4, C4)])
        plsc.subcore_barrier()

        def fetch_idx(j, slot):
            pltpu.async_copy(srcp.at[pl.ds(s * EPT + j * C4, C4)],
                             sidxb.at[slot], semi[slot])

        def body(j, _):
            slot = j % 2
            other = 1 - slot

            @pl.when(j < NCH4)
            def _():
                for sl in range(2):
                    @pl.when(slot == sl)
                    def _():
                        pltpu.make_async_copy(
                            srcp.at[pl.ds(0, C4)], sidxb.at[sl],
                            semi[sl]).wait()
                        for k in range(C4 // 16):
                            gidxb[sl, pl.ds(k * 16, 16)] = (
                                sidxb[sl, pl.ds(k * 16, 16)] + qoff)

                        @pl.when(j >= 2)
                        def _():
                            pltpu.make_async_copy(
                                rows.at[sl], acc.at[didx_all.at[0]],
                                sems[sl]).wait()

                        pltpu.async_copy(zq.at[gidxb.at[sl]], rows.at[sl],
                                        semr[sl])

                        @pl.when(j + 1 < NCH4)
                        def _():
                            fetch_idx(j + 1, 1 - sl)

            @pl.when(j > 0)
            def _():
                jm = j - 1
                for sl in range(2):
                    @pl.when(other == sl)
                    def _():
                        pltpu.make_async_copy(zq.at[gidxb.at[sl]],
                                              rows.at[sl], semr[sl]).wait()

                        def g_body(g, _):
                            av = alph_all[pl.ds(jm * C4 + g * 16, 16)]
                            for i in range(16):
                                e = g * 16 + i
                                a = av[i]
                                for k in range(8):
                                    rows[sl, e, pl.ds(k * 16, 16)] = (
                                        rows[sl, e, pl.ds(k * 16, 16)] * a)
                            return 0

                        lax.fori_loop(0, C4 // 16, g_body, 0)
                        pltpu.async_copy(rows.at[sl], acc.at[didx_all.at[jm]],
                                         sems[sl], add=True)
            return 0

        fetch_idx(0, 0)
        lax.fori_loop(0, NCH4 + 1, body, 0)
        for sl in range(2):
            pltpu.make_async_copy(rows.at[sl], acc.at[didx_all.at[0]],
                                  sems[sl]).wait()
        plsc.subcore_barrier()
        pltpu.sync_copy(acc.at[pl.ds(s * ROWS_PER_TILE, ROWS_PER_TILE)],
                        out.at[2 * p + c, pl.ds(s * ROWS_PER_TILE,
                                                ROWS_PER_TILE)])
        plsc.subcore_barrier()


_k4 = functools.partial(
    pl.kernel,
    out_type=jax.ShapeDtypeStruct((4, NP, 128), jnp.float32),
    mesh=_mesh,
    scratch_types=[
        pltpu.VMEM((NCH4, C4), jnp.int32),
        pltpu.VMEM((EPT,), jnp.float32),
        pltpu.VMEM((2, C4), jnp.int32),
        pltpu.VMEM((2, C4), jnp.int32),
        pltpu.VMEM((2, C4, 128), jnp.float32),
        pltpu.VMEM_SHARED((NP, 128), jnp.float32),
        pltpu.SemaphoreType.DMA,
        pltpu.SemaphoreType.DMA,
        pltpu.SemaphoreType.DMA,
        pltpu.SemaphoreType.DMA,
        pltpu.SemaphoreType.DMA,
        pltpu.SemaphoreType.DMA,
    ],
    compiler_params=pltpu.CompilerParams(needs_layout_passes=False),
)(_k4_body)


def _k5_body(zagg_ref, w4_ref, b_ref, o_ref):
    acc = jnp.broadcast_to(b_ref[0], (2000, HID))
    for q in range(4):
        acc = acc + lax.dot_general(zagg_ref[q], w4_ref[q],
                                    (((1,), (1,)), ((), ())),
                                    preferred_element_type=jnp.float32)
    o_ref[...] = acc


def _k5(zagg, w4, b):
    return pl.pallas_call(
        _k5_body,
        grid=(5,),
        in_specs=[
            pl.BlockSpec((4, 2000, 128), lambda i: (0, i, 0)),
            pl.BlockSpec((4, HID, 128), lambda i: (0, 0, 0)),
            pl.BlockSpec((1, HID), lambda i: (0, 0)),
        ],
        out_specs=pl.BlockSpec((2000, HID), lambda i: (i, 0)),
        out_shape=jax.ShapeDtypeStruct((N_NODES, HID), jnp.float32),
    )(zagg, w4, b)


def kernel(feat, edge_index, WC, WD, proj_W, proj_b):
    src = edge_index[0].astype(jnp.int32)
    dst = edge_index[1].astype(jnp.int32)
    pad = jnp.zeros((EP - N_EDGES,), jnp.int32)
    srcp = jnp.concatenate([src, pad])
    dstp = jnp.concatenate([dst, pad])

    wcat4 = jnp.concatenate([WC, WD], axis=0).reshape(4, 128, IN_CH)
    zq = _k1(feat, wcat4)
    zq2 = zq.reshape(4 * N_NODES, 128)

    sco, sdi = _k2(zq2, srcp.reshape(EP // C2, C2), dstp.reshape(EP // C2, C2))
    alpha_c, alpha_d = _k3(sco, sdi)
    zagg = _k4(zq2, srcp, dstp.reshape(EP // C4, C4), alpha_c, alpha_d)

    w4 = proj_W.reshape(HID, 4, 128).transpose(1, 0, 2)
    b2 = proj_b.reshape(1, HID)
    return _k5(zagg, w4, b2)


# speedup vs baseline: 2.4765x; 1.6298x over previous
import functools

import jax
import jax.numpy as jnp
from jax import lax
from jax.experimental import pallas as pl
from jax.experimental.pallas import tpu as pltpu
from jax.experimental.pallas import tpu_sc as plsc

N_NODES = 10000
N_EDGES = 160000
IN_CH = 256
HID = 256

NC = 2
NS = 16
NW = NC * NS

EP = 163840
EPW = EP // NW
C2 = 32
NCH2 = EPW // C2

EPT = EP // NS
C4 = 64
NCH4 = EPT // C4

NP = 10240
ROWS_PER_TILE = NP // NS
ZROWS = 128

_mesh = plsc.VectorSubcoreMesh(core_axis_name="c", subcore_axis_name="s")


def _k1_body(feat_ref, w_ref, zq_ref):
    zq_ref[0] = lax.dot_general(feat_ref[...], w_ref[0],
                                (((1,), (1,)), ((), ())),
                                preferred_element_type=jnp.float32)


def _k1(feat, wcat4):
    return pl.pallas_call(
        _k1_body,
        grid=(5, 4),
        in_specs=[
            pl.BlockSpec((2000, IN_CH), lambda i, q: (i, 0)),
            pl.BlockSpec((1, 128, IN_CH), lambda i, q: (q, 0, 0)),
        ],
        out_specs=pl.BlockSpec((1, 2000, 128), lambda i, q: (q, i, 0)),
        out_shape=jax.ShapeDtypeStruct((4, NP, 128), jnp.float32),
    )(feat, wcat4)


def _k2_body(zq, srcp2d, dstp2d, sco_hbm, sdi_hbm, sidxb, didxb, srows,
             drows, sco, sdi, zsh, semi0, semi1, semr0, semr1):
    s = lax.axis_index("s")
    wid = lax.axis_index("c") * NS + s
    wbase = wid * EPW
    lane = lax.broadcasted_iota(jnp.int32, (16,), 0)
    semi = (semi0, semi1)
    semr = (semr0, semr1)

    def fetch_idx(j, slot):
        pltpu.async_copy(srcp2d.at[wid * NCH2 + j], sidxb.at[slot],
                         semi[slot])
        pltpu.async_copy(dstp2d.at[wid * NCH2 + j], didxb.at[slot],
                         semi[slot])

    for q in range(4):
        rpt = NP // NS
        pltpu.sync_copy(zq.at[pl.ds(q * NP + s * rpt, rpt)],
                        zsh.at[pl.ds(s * rpt, rpt)])
        plsc.subcore_barrier()

        def body(j, _):
            slot = j % 2
            other = 1 - slot

            @pl.when(j < NCH2)
            def _():
                for sl in range(2):
                    @pl.when(slot == sl)
                    def _():
                        pltpu.make_async_copy(srcp2d.at[0], sidxb.at[sl],
                                              semi[sl]).wait()
                        pltpu.make_async_copy(dstp2d.at[0], didxb.at[sl],
                                              semi[sl]).wait()
                        pltpu.async_copy(zsh.at[sidxb.at[sl]], srows.at[sl],
                                         semr[sl])
                        pltpu.async_copy(zsh.at[didxb.at[sl]], drows.at[sl],
                                         semr[sl])

                        @pl.when(j + 1 < NCH2)
                        def _():
                            fetch_idx(j + 1, 1 - sl)

            @pl.when(j > 0)
            def _():
                jm = j - 1
                for sl in range(2):
                    @pl.when(other == sl)
                    def _():
                        pltpu.make_async_copy(zsh.at[sidxb.at[sl]],
                                              srows.at[sl], semr[sl]).wait()
                        pltpu.make_async_copy(zsh.at[sidxb.at[sl]],
                                              drows.at[sl], semr[sl]).wait()

                        def group_body(g, _):
                            sc = jnp.zeros((16,), jnp.float32)
                            for i in range(16):
                                e = g * 16 + i
                                a0 = jnp.zeros((16,), jnp.float32)
                                a1 = jnp.zeros((16,), jnp.float32)
                                a2 = jnp.zeros((16,), jnp.float32)
                                a3 = jnp.zeros((16,), jnp.float32)
                                for k in range(8):
                                    ks = pl.ds(k * 16, 16)
                                    sv = srows[sl, e, ks]
                                    dv = drows[sl, e, ks]
                                    if q < 2:
                                        pr = sv * dv
                                    else:
                                        df = sv - dv
                                        pr = df * df
                                    if k % 4 == 0:
                                        a0 = a0 + pr
                                    elif k % 4 == 1:
                                        a1 = a1 + pr
                                    elif k % 4 == 2:
                                        a2 = a2 + pr
                                    else:
                                        a3 = a3 + pr
                                acc = (a0 + a1) + (a2 + a3)
                                sc = jnp.where(lane == i, jnp.sum(acc), sc)
                            es = pl.ds(jm * C2 + g * 16, 16)
                            if q == 0:
                                sco[es] = sc
                            elif q == 1:
                                sco[es] = sco[es] + sc
                            elif q == 2:
                                sdi[es] = sc
                            else:
                                sdi[es] = sdi[es] + sc
                            return 0

                        lax.fori_loop(0, C2 // 16, group_body, 0)
            return 0

        fetch_idx(0, 0)
        lax.fori_loop(0, NCH2 + 1, body, 0)
        plsc.subcore_barrier()

    pltpu.sync_copy(sco, sco_hbm.at[pl.ds(wbase, EPW)])
    pltpu.sync_copy(sdi, sdi_hbm.at[pl.ds(wbase, EPW)])


_k2 = functools.partial(
    pl.kernel,
    out_type=(jax.ShapeDtypeStruct((EP,), jnp.float32),
              jax.ShapeDtypeStruct((EP,), jnp.float32)),
    mesh=_mesh,
    scratch_types=[
        pltpu.VMEM((2, C2), jnp.int32),
        pltpu.VMEM((2, C2), jnp.int32),
        pltpu.VMEM((2, C2, 128), jnp.float32),
        pltpu.VMEM((2, C2, 128), jnp.float32),
        pltpu.VMEM((EPW,), jnp.float32),
        pltpu.VMEM((EPW,), jnp.float32),
        pltpu.VMEM_SHARED((NP, 128), jnp.float32),
        pltpu.SemaphoreType.DMA,
        pltpu.SemaphoreType.DMA,
        pltpu.SemaphoreType.DMA,
        pltpu.SemaphoreType.DMA,
    ],
    compiler_params=pltpu.CompilerParams(needs_layout_passes=False),
)(_k2_body)


def _k3_body(co_ref, di_ref, ao_ref, ad_ref):
    row = lax.broadcasted_iota(jnp.int32, (EP // 128, 128), 0)
    valid = row < (N_EDGES // 128)
    for ref, oref in ((co_ref, ao_ref), (di_ref, ad_ref)):
        x = ref[...]
        xm = jnp.where(valid, x, -jnp.inf)
        m = jnp.max(xm)
        ex = jnp.where(valid, jnp.exp(x - m), 0.0)
        oref[...] = ex / jnp.sum(ex)


def _k3(sco, sdi):
    out = pl.pallas_call(
        _k3_body,
        out_shape=[jax.ShapeDtypeStruct((EP // 128, 128), jnp.float32),
                   jax.ShapeDtypeStruct((EP // 128, 128), jnp.float32)],
    )(sco.reshape(EP // 128, 128), sdi.reshape(EP // 128, 128))
    return out[0].reshape(EP), out[1].reshape(EP)


def _k4_body(zq, srcp, dstp2d, alpha_c, alpha_d, out, didx_all, alph_all,
             sidxb, gidxb, rows, acc, semi0, semi1, semr0, semr1, sems0,
             sems1):
    c = lax.axis_index("c")
    s = lax.axis_index("s")
    pltpu.sync_copy(dstp2d.at[pl.ds(s * NCH4, NCH4)], didx_all)
    semi = (semi0, semi1)
    semr = (semr0, semr1)
    sems = (sems0, sems1)

    for p, alpha in ((0, alpha_c), (1, alpha_d)):
        qoff = (2 * p + c) * NP
        pltpu.sync_copy(alpha.at[pl.ds(s * EPT, EPT)], alph_all)

        def zb_body(r, _):
            for k in range(8):
                rows[0, r, pl.ds(k * 16, 16)] = jnp.zeros((16,), jnp.float32)
            return 0

        lax.fori_loop(0, C4, zb_body, 0)
        for i in range(ROWS_PER_TILE // C4):
            pltpu.sync_copy(rows.at[0], acc.at[pl.ds(s * ROWS_PER_TILE
                                                     + i * C4, C4)])
        plsc.subcore_barrier()

        def fetch_idx(j, slot):
            pltpu.async_copy(srcp.at[pl.ds(s * EPT + j * C4, C4)],
                             sidxb.at[slot], semi[slot])

        def body(j, _):
            slot = j % 2
            other = 1 - slot

            @pl.when(j < NCH4)
            def _():
                for sl in range(2):
                    @pl.when(slot == sl)
                    def _():
                        pltpu.make_async_copy(
                            srcp.at[pl.ds(0, C4)], sidxb.at[sl],
                            semi[sl]).wait()
                        for k in range(C4 // 16):
                            gidxb[sl, pl.ds(k * 16, 16)] = (
                                sidxb[sl, pl.ds(k * 16, 16)] + qoff)

                        @pl.when(j >= 2)
                        def _():
                            pltpu.make_async_copy(
                                rows.at[sl], acc.at[didx_all.at[0]],
                                sems[sl]).wait()

                        pltpu.async_copy(zq.at[gidxb.at[sl]], rows.at[sl],
                                        semr[sl])

                        @pl.when(j + 1 < NCH4)
                        def _():
                            fetch_idx(j + 1, 1 - sl)

            @pl.when(j > 0)
            def _():
                jm = j - 1
                for sl in range(2):
                    @pl.when(other == sl)
                    def _():
                        pltpu.make_async_copy(zq.at[gidxb.at[sl]],
                                              rows.at[sl], semr[sl]).wait()

                        def g_body(g, _):
                            av = alph_all[pl.ds(jm * C4 + g * 16, 16)]
                            for i in range(16):
                                e = g * 16 + i
                                a = av[i]
                                for k in range(8):
                                    rows[sl, e, pl.ds(k * 16, 16)] = (
                                        rows[sl, e, pl.ds(k * 16, 16)] * a)
                            return 0

                        lax.fori_loop(0, C4 // 16, g_body, 0)
                        pltpu.async_copy(rows.at[sl], acc.at[didx_all.at[jm]],
                                         sems[sl], add=True)
            return 0

        fetch_idx(0, 0)
        lax.fori_loop(0, NCH4 + 1, body, 0)
        for sl in range(2):
            pltpu.make_async_copy(rows.at[sl], acc.at[didx_all.at[0]],
                                  sems[sl]).wait()
        plsc.subcore_barrier()
        pltpu.sync_copy(acc.at[pl.ds(s * ROWS_PER_TILE, ROWS_PER_TILE)],
                        out.at[2 * p + c, pl.ds(s * ROWS_PER_TILE,
                                                ROWS_PER_TILE)])
        plsc.subcore_barrier()


_k4 = functools.partial(
    pl.kernel,
    out_type=jax.ShapeDtypeStruct((4, NP, 128), jnp.float32),
    mesh=_mesh,
    scratch_types=[
        pltpu.VMEM((NCH4, C4), jnp.int32),
        pltpu.VMEM((EPT,), jnp.float32),
        pltpu.VMEM((2, C4), jnp.int32),
        pltpu.VMEM((2, C4), jnp.int32),
        pltpu.VMEM((2, C4, 128), jnp.float32),
        pltpu.VMEM_SHARED((NP, 128), jnp.float32),
        pltpu.SemaphoreType.DMA,
        pltpu.SemaphoreType.DMA,
        pltpu.SemaphoreType.DMA,
        pltpu.SemaphoreType.DMA,
        pltpu.SemaphoreType.DMA,
        pltpu.SemaphoreType.DMA,
    ],
    compiler_params=pltpu.CompilerParams(needs_layout_passes=False),
)(_k4_body)


def _k5_body(zagg_ref, w4_ref, b_ref, o_ref):
    acc = jnp.broadcast_to(b_ref[0], (2000, HID))
    for q in range(4):
        acc = acc + lax.dot_general(zagg_ref[q], w4_ref[q],
                                    (((1,), (1,)), ((), ())),
                                    preferred_element_type=jnp.float32)
    o_ref[...] = acc


def _k5(zagg, w4, b):
    return pl.pallas_call(
        _k5_body,
        grid=(5,),
        in_specs=[
            pl.BlockSpec((4, 2000, 128), lambda i: (0, i, 0)),
            pl.BlockSpec((4, HID, 128), lambda i: (0, 0, 0)),
            pl.BlockSpec((1, HID), lambda i: (0, 0)),
        ],
        out_specs=pl.BlockSpec((2000, HID), lambda i: (i, 0)),
        out_shape=jax.ShapeDtypeStruct((N_NODES, HID), jnp.float32),
    )(zagg, w4, b)


def kernel(feat, edge_index, WC, WD, proj_W, proj_b):
    src = edge_index[0].astype(jnp.int32)
    dst = edge_index[1].astype(jnp.int32)
    pad = jnp.zeros((EP - N_EDGES,), jnp.int32)
    srcp = jnp.concatenate([src, pad])
    dstp = jnp.concatenate([dst, pad])

    wcat4 = jnp.concatenate([WC, WD], axis=0).reshape(4, 128, IN_CH)
    zq = _k1(feat, wcat4)
    zq2 = zq.reshape(4 * NP, 128)

    sco, sdi = _k2(zq2, srcp.reshape(EP // C2, C2), dstp.reshape(EP // C2, C2))
    alpha_c, alpha_d = _k3(sco, sdi)
    zagg = _k4(zq2, srcp, dstp.reshape(EP // C4, C4), alpha_c, alpha_d)

    w4 = proj_W.reshape(HID, 4, 128).transpose(1, 0, 2)
    b2 = proj_b.reshape(1, HID)
    return _k5(zagg, w4, b2)
